# Initial kernel scaffold; baseline (speedup 1.0000x reference)
import jax
import jax.numpy as jnp
from jax.experimental import pallas as pl


def kernel(img, index, max_index):
    raise NotImplementedError("write your pallas kernel here")


# speedup vs baseline: 1.1393x; 1.1393x over previous
import functools

import jax
import jax.numpy as jnp
from jax import lax
from jax.experimental import pallas as pl
from jax.experimental.pallas import tpu as pltpu
from jax.experimental.pallas import tpu_sc as plsc

B = 4
F = 192
HW = 224 * 224
S = 512
L = 16
NC, NS = 2, 16
NW = NC * NS
TPB = NW // B
FPT = F // TPB
CHUNK = HW // 4
NCH = HW // CHUNK
ACC = S * L + L


def _body(img_hbm, idx_hbm, out_hbm, addr_v, acc_a, acc_b, buf0, buf1,
          out_v, sem0, sem1):
    wid = lax.axis_index("s") * NC + lax.axis_index("c")
    b = wid // TPB
    f0 = (wid % TPB) * FPT
    lane = lax.broadcasted_iota(jnp.int32, (L,), 0)

    pltpu.sync_copy(idx_hbm.at[b], addr_v)

    def mk_addr(i, c):
        v = addr_v[pl.ds(i * L, L)]
        v = jnp.minimum(v, S)
        addr_v[pl.ds(i * L, L)] = jnp.where(
            v == 0, S * L + lane, lane * S + (v - 1))
        return c

    lax.fori_loop(0, HW // L, mk_addr, 0)

    bufs = (buf0, buf1)
    sems = (sem0, sem1)

    def src(f, part):
        return img_hbm.at[b, f0 + f, pl.ds(part * CHUNK, CHUNK)]

    pltpu.async_copy(src(0, 0), buf0, sem0)

    def feature_body(f, c):
        def zero(i, cc):
            z = jnp.zeros((L,), jnp.float32)
            acc_a[pl.ds(i * L, L)] = z
            acc_b[pl.ds(i * L, L)] = z
            return cc

        lax.fori_loop(0, ACC // L, zero, c)

        for part in range(NCH):
            cur, csem = bufs[part % 2], sems[part % 2]
            if part + 1 < NCH:
                pltpu.async_copy(src(f, part + 1), bufs[(part + 1) % 2],
                                 sems[(part + 1) % 2])
            else:
                fn = jnp.minimum(f + 1, FPT - 1)
                pltpu.async_copy(src(fn, 0), buf0, sem0)
            pltpu.make_async_copy(src(f, part), cur, csem).wait()

            po = part * CHUNK

            def scat(i, cc):
                base = i * (2 * L)
                a0 = addr_v[pl.ds(po + base, L)]
                v0 = cur[pl.ds(base, L)]
                g0 = plsc.load_gather(acc_a, [a0])
                plsc.store_scatter(acc_a, [a0], jnp.maximum(g0, v0))
                a1 = addr_v[pl.ds(po + base + L, L)]
                v1 = cur[pl.ds(base + L, L)]
                g1 = plsc.load_gather(acc_b, [a1])
                plsc.store_scatter(acc_b, [a1], jnp.maximum(g1, v1))
                return cc

            lax.fori_loop(0, CHUNK // (2 * L), scat, c)

        def red(j, cc):
            s0 = j * L
            m = acc_a[pl.ds(s0, L)]
            m = jnp.maximum(m, acc_b[pl.ds(s0, L)])
            for l in range(1, L):
                m = jnp.maximum(m, acc_a[pl.ds(l * S + s0, L)])
                m = jnp.maximum(m, acc_b[pl.ds(l * S + s0, L)])
            out_v[pl.ds(s0, L)] = m
            return cc

        lax.fori_loop(0, S // L, red, c)
        pltpu.sync_copy(out_v, out_hbm.at[b, f0 + f])
        return c

    lax.fori_loop(0, FPT, feature_body, 0)
    pltpu.make_async_copy(src(0, 0), buf0, sem0).wait()


@jax.jit
def _face_pool(img3, idx2):
    mesh = plsc.VectorSubcoreMesh(core_axis_name="c", subcore_axis_name="s")
    return pl.kernel(
        _body,
        out_type=jax.ShapeDtypeStruct((B, F, S), jnp.float32),
        mesh=mesh,
        compiler_params=pltpu.CompilerParams(needs_layout_passes=False),
        scratch_types=[
            pltpu.VMEM((HW,), jnp.int32),
            pltpu.VMEM((ACC,), jnp.float32),
            pltpu.VMEM((ACC,), jnp.float32),
            pltpu.VMEM((CHUNK,), jnp.float32),
            pltpu.VMEM((CHUNK,), jnp.float32),
            pltpu.VMEM((S,), jnp.float32),
            pltpu.SemaphoreType.DMA,
            pltpu.SemaphoreType.DMA,
        ],
    )(img3, idx2)


def kernel(img, index, max_index):
    del max_index
    b, f, h, w = img.shape
    img3 = img.reshape(b, f, h * w)
    idx2 = index.reshape(b, h * w).astype(jnp.int32)
    return _face_pool(img3, idx2)


# speedup vs baseline: 1.5650x; 1.3737x over previous
import functools

import jax
import jax.numpy as jnp
from jax import lax
from jax.experimental import pallas as pl
from jax.experimental.pallas import tpu as pltpu
from jax.experimental.pallas import tpu_sc as plsc

B = 4
F = 192
HW = 224 * 224
S = 512
L = 16
NC, NS = 2, 16
NW = NC * NS
TPB = NW // B
FPT = F // TPB
K = 4
GRP = FPT // K
CH = 3584
NCH = HW // CH
ACC = S * L + L


def _body(img_hbm, idx_hbm, out_hbm, addr_v, acc0, acc1, acc2, acc3,
          buf0, buf1, out_v, sem0, sem1):
    accs = (acc0, acc1, acc2, acc3)
    bufs = (buf0, buf1)
    sems = (sem0, sem1)
    wid = lax.axis_index("s") * NC + lax.axis_index("c")
    b = wid // TPB
    f0 = (wid % TPB) * FPT
    lane = lax.broadcasted_iota(jnp.int32, (L,), 0)

    pltpu.sync_copy(idx_hbm.at[b], addr_v)

    def mk_addr(i, c):
        v = addr_v[pl.ds(i * L, L)]
        v = jnp.minimum(v, S)
        addr_v[pl.ds(i * L, L)] = jnp.where(
            v == 0, S * L + lane, lane * S + (v - 1))
        return c

    lax.fori_loop(0, HW // L, mk_addr, 0)

    def src(g, c):
        return img_hbm.at[b, pl.ds(f0 + g * K, K), pl.ds(c * CH, CH)]

    pltpu.async_copy(src(0, 0), buf0, sem0)

    def group_body(g, carry):
        def zero(i, cc):
            z = jnp.zeros((L,), jnp.float32)
            for acc in accs:
                acc[pl.ds(i * L, L)] = z
            return cc

        lax.fori_loop(0, ACC // L, zero, carry)

        for c in range(NCH):
            cur, csem = bufs[c % 2], sems[c % 2]
            if c + 1 < NCH:
                pltpu.async_copy(src(g, c + 1), bufs[(c + 1) % 2],
                                 sems[(c + 1) % 2])
            else:
                gn = jnp.minimum(g + 1, GRP - 1)
                pltpu.async_copy(src(gn, 0), buf0, sem0)
            pltpu.make_async_copy(src(g, c), cur, csem).wait()

            po = c * CH

            def scat(i, cc):
                for u in range(2):
                    base = i * (2 * L) + u * L
                    ad = addr_v[pl.ds(po + base, L)]
                    for k in range(K):
                        v = cur[k, pl.ds(base, L)]
                        gk = plsc.load_gather(accs[k], [ad])
                        plsc.store_scatter(accs[k], [ad],
                                           jnp.maximum(gk, v))
                return cc

            lax.fori_loop(0, CH // (2 * L), scat, carry)

        for k in range(K):
            def red(j, cc, *, acc=accs[k]):
                s0 = j * L
                m = acc[pl.ds(s0, L)]
                for l in range(1, L):
                    m = jnp.maximum(m, acc[pl.ds(l * S + s0, L)])
                out_v[pl.ds(s0, L)] = m
                return cc

            lax.fori_loop(0, S // L, red, carry)
            pltpu.sync_copy(out_v, out_hbm.at[b, f0 + g * K + k])
        return carry

    lax.fori_loop(0, GRP, group_body, 0)
    pltpu.make_async_copy(src(0, 0), buf0, sem0).wait()


@jax.jit
def _face_pool(img3, idx2):
    mesh = plsc.VectorSubcoreMesh(core_axis_name="c", subcore_axis_name="s")
    return pl.kernel(
        _body,
        out_type=jax.ShapeDtypeStruct((B, F, S), jnp.float32),
        mesh=mesh,
        compiler_params=pltpu.CompilerParams(needs_layout_passes=False),
        scratch_types=[
            pltpu.VMEM((HW,), jnp.int32),
            pltpu.VMEM((ACC,), jnp.float32),
            pltpu.VMEM((ACC,), jnp.float32),
            pltpu.VMEM((ACC,), jnp.float32),
            pltpu.VMEM((ACC,), jnp.float32),
            pltpu.VMEM((K, CH), jnp.float32),
            pltpu.VMEM((K, CH), jnp.float32),
            pltpu.VMEM((S,), jnp.float32),
            pltpu.SemaphoreType.DMA,
            pltpu.SemaphoreType.DMA,
        ],
    )(img3, idx2)


def kernel(img, index, max_index):
    del max_index
    b, f, h, w = img.shape
    img3 = img.reshape(b, f, h * w)
    idx2 = index.reshape(b, h * w).astype(jnp.int32)
    return _face_pool(img3, idx2)


# speedup vs baseline: 1.6609x; 1.0613x over previous
import functools

import jax
import jax.numpy as jnp
from jax import lax
from jax.experimental import pallas as pl
from jax.experimental.pallas import tpu as pltpu
from jax.experimental.pallas import tpu_sc as plsc

B = 4
F = 192
HW = 224 * 224
S = 512
L = 16
NC, NS = 2, 16
NW = NC * NS
TPB = NW // B
FPT = F // TPB
K = 4
GRP = FPT // K
CH = 896
NCH = HW // CH
ACC = S * L + L


def _body(img_hbm, idx_hbm, out_hbm, addr_v,
          a0, a1, a2, a3, b0, b1, b2, b3,
          buf0, buf1, out_v, sem0, sem1):
    acc_a = (a0, a1, a2, a3)
    acc_b = (b0, b1, b2, b3)
    wid = lax.axis_index("s") * NC + lax.axis_index("c")
    b = wid // TPB
    f0 = (wid % TPB) * FPT
    lane = lax.broadcasted_iota(jnp.int32, (L,), 0)

    pltpu.sync_copy(idx_hbm.at[b], addr_v)

    def mk_addr(i, c):
        v = addr_v[pl.ds(i * L, L)]
        v = jnp.minimum(v, S)
        addr_v[pl.ds(i * L, L)] = jnp.where(
            v == 0, S * L + lane, lane * S + (v - 1))
        return c

    lax.fori_loop(0, HW // L, mk_addr, 0)

    def src(g, c):
        return img_hbm.at[b, pl.ds(f0 + g * K, K), pl.ds(c * CH, CH)]

    def process(cur, po, carry):
        def scat(i, cc):
            base = i * (2 * L)
            ad0 = addr_v[pl.ds(po + base, L)]
            ad1 = addr_v[pl.ds(po + base + L, L)]
            for k in range(K):
                v = cur[k, pl.ds(base, L)]
                gk = plsc.load_gather(acc_a[k], [ad0])
                plsc.store_scatter(acc_a[k], [ad0], jnp.maximum(gk, v))
            for k in range(K):
                v = cur[k, pl.ds(base + L, L)]
                gk = plsc.load_gather(acc_b[k], [ad1])
                plsc.store_scatter(acc_b[k], [ad1], jnp.maximum(gk, v))
            return cc

        return lax.fori_loop(0, CH // (2 * L), scat, carry)

    pltpu.async_copy(src(0, 0), buf0, sem0)
    pltpu.async_copy(src(0, 1), buf1, sem1)

    def group_body(g, carry):
        def zero(i, cc):
            z = jnp.zeros((L,), jnp.float32)
            for acc in acc_a + acc_b:
                acc[pl.ds(i * L, L)] = z
            return cc

        carry = lax.fori_loop(0, ACC // L, zero, carry)

        def pair(c2, cc):
            c = c2 * 2
            pltpu.make_async_copy(src(g, c), buf0, sem0).wait()
            cc = process(buf0, c * CH, cc)
            pltpu.async_copy(src(g, c + 2), buf0, sem0)
            pltpu.make_async_copy(src(g, c + 1), buf1, sem1).wait()
            cc = process(buf1, (c + 1) * CH, cc)
            pltpu.async_copy(src(g, c + 3), buf1, sem1)
            return cc

        carry = lax.fori_loop(0, NCH // 2 - 1, pair, carry)

        gn = jnp.minimum(g + 1, GRP - 1)
        pltpu.make_async_copy(src(g, NCH - 2), buf0, sem0).wait()
        carry = process(buf0, (NCH - 2) * CH, carry)
        pltpu.async_copy(src(gn, 0), buf0, sem0)
        pltpu.make_async_copy(src(g, NCH - 1), buf1, sem1).wait()
        carry = process(buf1, (NCH - 1) * CH, carry)
        pltpu.async_copy(src(gn, 1), buf1, sem1)

        for k in range(K):
            def red(j, cc, *, ka=acc_a[k], kb=acc_b[k]):
                s0 = j * L
                m = jnp.maximum(ka[pl.ds(s0, L)], kb[pl.ds(s0, L)])
                for l in range(1, L):
                    m = jnp.maximum(m, ka[pl.ds(l * S + s0, L)])
                    m = jnp.maximum(m, kb[pl.ds(l * S + s0, L)])
                out_v[pl.ds(s0, L)] = m
                return cc

            carry = lax.fori_loop(0, S // L, red, carry)
            pltpu.sync_copy(out_v, out_hbm.at[b, f0 + g * K + k])
        return carry

    lax.fori_loop(0, GRP, group_body, 0)
    pltpu.make_async_copy(src(0, 0), buf0, sem0).wait()
    pltpu.make_async_copy(src(0, 1), buf1, sem1).wait()


@jax.jit
def _face_pool(img3, idx2):
    mesh = plsc.VectorSubcoreMesh(core_axis_name="c", subcore_axis_name="s")
    return pl.kernel(
        _body,
        out_type=jax.ShapeDtypeStruct((B, F, S), jnp.float32),
        mesh=mesh,
        compiler_params=pltpu.CompilerParams(needs_layout_passes=False),
        scratch_types=[
            pltpu.VMEM((HW,), jnp.int32),
            pltpu.VMEM((ACC,), jnp.float32),
            pltpu.VMEM((ACC,), jnp.float32),
            pltpu.VMEM((ACC,), jnp.float32),
            pltpu.VMEM((ACC,), jnp.float32),
            pltpu.VMEM((ACC,), jnp.float32),
            pltpu.VMEM((ACC,), jnp.float32),
            pltpu.VMEM((ACC,), jnp.float32),
            pltpu.VMEM((ACC,), jnp.float32),
            pltpu.VMEM((K, CH), jnp.float32),
            pltpu.VMEM((K, CH), jnp.float32),
            pltpu.VMEM((S,), jnp.float32),
            pltpu.SemaphoreType.DMA,
            pltpu.SemaphoreType.DMA,
        ],
    )(img3, idx2)


def kernel(img, index, max_index):
    del max_index
    b, f, h, w = img.shape
    img3 = img.reshape(b, f, h * w)
    idx2 = index.reshape(b, h * w).astype(jnp.int32)
    return _face_pool(img3, idx2)


# speedup vs baseline: 1.7782x; 1.0706x over previous
import functools

import jax
import jax.numpy as jnp
from jax import lax
from jax.experimental import pallas as pl
from jax.experimental.pallas import tpu as pltpu
from jax.experimental.pallas import tpu_sc as plsc

B = 4
F = 192
HW = 224 * 224
S = 512
L = 16
NC, NS = 2, 16
NW = NC * NS
TPB = NW // B
FPT = F // TPB
K = 8
GRP = FPT // K
CH = 256
NCH = HW // CH
ACC = S * L + L


def _body(img_hbm, idx_hbm, out_hbm, addr_v,
          a0, a1, a2, a3, a4, a5, a6, a7,
          buf0, buf1, out_v, sem0, sem1):
    accs = (a0, a1, a2, a3, a4, a5, a6, a7)
    wid = lax.axis_index("s") * NC + lax.axis_index("c")
    b = wid // TPB
    f0 = (wid % TPB) * FPT
    lane = lax.broadcasted_iota(jnp.int32, (L,), 0)

    pltpu.sync_copy(idx_hbm.at[b], addr_v)

    def mk_addr(i, c):
        v = addr_v[pl.ds(i * L, L)]
        v = jnp.minimum(v, S)
        addr_v[pl.ds(i * L, L)] = jnp.where(
            v == 0, S * L + lane, (v - 1) * L + lane)
        return c

    lax.fori_loop(0, HW // L, mk_addr, 0)

    def src(g, c):
        return img_hbm.at[b, pl.ds(f0 + g * K, K), pl.ds(c * CH, CH)]

    def process(cur, po, carry):
        def scat(i, cc):
            base = i * L
            ad = addr_v[pl.ds(po + base, L)]
            for k in range(K):
                v = cur[k, pl.ds(base, L)]
                gk = plsc.load_gather(accs[k], [ad])
                plsc.store_scatter(accs[k], [ad], jnp.maximum(gk, v))
            return cc

        return lax.fori_loop(0, CH // L, scat, carry)

    perm_idx = tuple(jnp.bitwise_xor(lane, d) for d in (8, 4, 2, 1))
    lane_bit = tuple((lane & d) == 0 for d in (8, 4, 2, 1))

    pltpu.async_copy(src(0, 0), buf0, sem0)
    pltpu.async_copy(src(0, 1), buf1, sem1)

    def group_body(g, carry):
        def zero(i, cc):
            z = jnp.zeros((L,), jnp.float32)
            for acc in accs:
                acc[pl.ds(i * L, L)] = z
            return cc

        carry = lax.fori_loop(0, ACC // L, zero, carry)

        def pair(c2, cc):
            c = c2 * 2
            pltpu.make_async_copy(src(g, c), buf0, sem0).wait()
            cc = process(buf0, c * CH, cc)
            pltpu.async_copy(src(g, c + 2), buf0, sem0)
            pltpu.make_async_copy(src(g, c + 1), buf1, sem1).wait()
            cc = process(buf1, (c + 1) * CH, cc)
            pltpu.async_copy(src(g, c + 3), buf1, sem1)
            return cc

        carry = lax.fori_loop(0, NCH // 2 - 1, pair, carry)

        gn = jnp.minimum(g + 1, GRP - 1)
        pltpu.make_async_copy(src(g, NCH - 2), buf0, sem0).wait()
        carry = process(buf0, (NCH - 2) * CH, carry)
        pltpu.async_copy(src(gn, 0), buf0, sem0)
        pltpu.make_async_copy(src(g, NCH - 1), buf1, sem1).wait()
        carry = process(buf1, (NCH - 1) * CH, carry)
        pltpu.async_copy(src(gn, 1), buf1, sem1)

        for k in range(K):
            def red(j, cc, *, acc=accs[k]):
                rows = [acc[pl.ds(j * (L * L) + i * L, L)]
                        for i in range(L)]
                for lvl, d in enumerate((8, 4, 2, 1)):
                    half = len(rows) // 2
                    nxt = []
                    for i in range(half):
                        va, vb = rows[i], rows[i + half]
                        pa = va.at[perm_idx[lvl]].get(
                            mode="promise_in_bounds")
                        pb = vb.at[perm_idx[lvl]].get(
                            mode="promise_in_bounds")
                        nxt.append(jnp.where(lane_bit[lvl],
                                             jnp.maximum(va, pa),
                                             jnp.maximum(vb, pb)))
                    rows = nxt
                out_v[pl.ds(j * L, L)] = rows[0]
                return cc

            carry = lax.fori_loop(0, S // L, red, carry)
            pltpu.sync_copy(out_v, out_hbm.at[b, f0 + g * K + k])
        return carry

    lax.fori_loop(0, GRP, group_body, 0)
    pltpu.make_async_copy(src(0, 0), buf0, sem0).wait()
    pltpu.make_async_copy(src(0, 1), buf1, sem1).wait()


@jax.jit
def _face_pool(img3, idx2):
    mesh = plsc.VectorSubcoreMesh(core_axis_name="c", subcore_axis_name="s")
    return pl.kernel(
        _body,
        out_type=jax.ShapeDtypeStruct((B, F, S), jnp.float32),
        mesh=mesh,
        compiler_params=pltpu.CompilerParams(needs_layout_passes=False),
        scratch_types=[
            pltpu.VMEM((HW,), jnp.int32),
            pltpu.VMEM((ACC,), jnp.float32),
            pltpu.VMEM((ACC,), jnp.float32),
            pltpu.VMEM((ACC,), jnp.float32),
            pltpu.VMEM((ACC,), jnp.float32),
            pltpu.VMEM((ACC,), jnp.float32),
            pltpu.VMEM((ACC,), jnp.float32),
            pltpu.VMEM((ACC,), jnp.float32),
            pltpu.VMEM((ACC,), jnp.float32),
            pltpu.VMEM((K, CH), jnp.float32),
            pltpu.VMEM((K, CH), jnp.float32),
            pltpu.VMEM((S,), jnp.float32),
            pltpu.SemaphoreType.DMA,
            pltpu.SemaphoreType.DMA,
        ],
    )(img3, idx2)


def kernel(img, index, max_index):
    del max_index
    b, f, h, w = img.shape
    img3 = img.reshape(b, f, h * w)
    idx2 = index.reshape(b, h * w).astype(jnp.int32)
    return _face_pool(img3, idx2)


# speedup vs baseline: 2.2789x; 1.2816x over previous
import functools

import jax
import jax.numpy as jnp
from jax import lax
from jax.experimental import pallas as pl
from jax.experimental.pallas import tpu as pltpu
from jax.experimental.pallas import tpu_sc as plsc

B = 4
F = 192
HW = 224 * 224
S = 512
L = 16
NC, NS = 2, 16
NW = NC * NS
TPB = NW // B
FPT = F // TPB
K = 8
GRP = FPT // K
CH = 256
NCH = HW // CH
ACC = S * L + L


def _body(img_hbm, idx_hbm, out_hbm, addr_v,
          a0, a1, a2, a3, a4, a5, a6, a7,
          buf0, buf1, out_v, sem0, sem1):
    accs = (a0, a1, a2, a3, a4, a5, a6, a7)
    wid = lax.axis_index("s") * NC + lax.axis_index("c")
    b = wid // TPB
    f0 = (wid % TPB) * FPT
    lane = lax.broadcasted_iota(jnp.int32, (L,), 0)

    pltpu.sync_copy(idx_hbm.at[b], addr_v)

    def mk_addr(i, c):
        v = addr_v[pl.ds(i * L, L)]
        v = jnp.minimum(v, S)
        addr_v[pl.ds(i * L, L)] = jnp.where(
            v == 0, S * L + lane, (v - 1) * L + lane)
        return c

    lax.fori_loop(0, HW // L, mk_addr, 0)

    def src(g, c):
        return img_hbm.at[b, pl.ds(f0 + g * K, K), pl.ds(c * CH, CH)]

    def process(cur, po, carry):
        def scat(i, cc):
            base = i * L
            ad = addr_v[pl.ds(po + base, L)]
            gs = [plsc.load_gather(accs[k], [ad]) for k in range(K)]
            vs = [cur[k, pl.ds(base, L)] for k in range(K)]
            ms = [jnp.maximum(g, v) for g, v in zip(gs, vs)]
            for k in range(K):
                plsc.store_scatter(accs[k], [ad], ms[k])
            return cc

        return lax.fori_loop(0, CH // L, scat, carry)

    perm_idx = tuple(jnp.bitwise_xor(lane, d) for d in (8, 4, 2, 1))
    lane_bit = tuple((lane & d) == 0 for d in (8, 4, 2, 1))

    pltpu.async_copy(src(0, 0), buf0, sem0)
    pltpu.async_copy(src(0, 1), buf1, sem1)

    def group_body(g, carry):
        def zero(i, cc):
            z = jnp.zeros((L,), jnp.float32)
            for acc in accs:
                acc[pl.ds(i * L, L)] = z
            return cc

        carry = lax.fori_loop(0, ACC // L, zero, carry)

        def pair(c2, cc):
            c = c2 * 2
            pltpu.make_async_copy(src(g, c), buf0, sem0).wait()
            cc = process(buf0, c * CH, cc)
            pltpu.async_copy(src(g, c + 2), buf0, sem0)
            pltpu.make_async_copy(src(g, c + 1), buf1, sem1).wait()
            cc = process(buf1, (c + 1) * CH, cc)
            pltpu.async_copy(src(g, c + 3), buf1, sem1)
            return cc

        carry = lax.fori_loop(0, NCH // 2 - 1, pair, carry)

        gn = jnp.minimum(g + 1, GRP - 1)
        pltpu.make_async_copy(src(g, NCH - 2), buf0, sem0).wait()
        carry = process(buf0, (NCH - 2) * CH, carry)
        pltpu.async_copy(src(gn, 0), buf0, sem0)
        pltpu.make_async_copy(src(g, NCH - 1), buf1, sem1).wait()
        carry = process(buf1, (NCH - 1) * CH, carry)
        pltpu.async_copy(src(gn, 1), buf1, sem1)

        for k in range(K):
            def red(j, cc, *, acc=accs[k]):
                rows = [acc[pl.ds(j * (L * L) + i * L, L)]
                        for i in range(L)]
                for lvl, d in enumerate((8, 4, 2, 1)):
                    half = len(rows) // 2
                    nxt = []
                    for i in range(half):
                        va, vb = rows[i], rows[i + half]
                        pa = va.at[perm_idx[lvl]].get(
                            mode="promise_in_bounds")
                        pb = vb.at[perm_idx[lvl]].get(
                            mode="promise_in_bounds")
                        nxt.append(jnp.where(lane_bit[lvl],
                                             jnp.maximum(va, pa),
                                             jnp.maximum(vb, pb)))
                    rows = nxt
                out_v[pl.ds(j * L, L)] = rows[0]
                return cc

            carry = lax.fori_loop(0, S // L, red, carry)
            pltpu.sync_copy(out_v, out_hbm.at[b, f0 + g * K + k])
        return carry

    lax.fori_loop(0, GRP, group_body, 0)
    pltpu.make_async_copy(src(0, 0), buf0, sem0).wait()
    pltpu.make_async_copy(src(0, 1), buf1, sem1).wait()


@jax.jit
def _face_pool(img3, idx2):
    mesh = plsc.VectorSubcoreMesh(core_axis_name="c", subcore_axis_name="s")
    return pl.kernel(
        _body,
        out_type=jax.ShapeDtypeStruct((B, F, S), jnp.float32),
        mesh=mesh,
        compiler_params=pltpu.CompilerParams(needs_layout_passes=False),
        scratch_types=[
            pltpu.VMEM((HW,), jnp.int32),
            pltpu.VMEM((ACC,), jnp.float32),
            pltpu.VMEM((ACC,), jnp.float32),
            pltpu.VMEM((ACC,), jnp.float32),
            pltpu.VMEM((ACC,), jnp.float32),
            pltpu.VMEM((ACC,), jnp.float32),
            pltpu.VMEM((ACC,), jnp.float32),
            pltpu.VMEM((ACC,), jnp.float32),
            pltpu.VMEM((ACC,), jnp.float32),
            pltpu.VMEM((K, CH), jnp.float32),
            pltpu.VMEM((K, CH), jnp.float32),
            pltpu.VMEM((S,), jnp.float32),
            pltpu.SemaphoreType.DMA,
            pltpu.SemaphoreType.DMA,
        ],
    )(img3, idx2)


def kernel(img, index, max_index):
    del max_index
    b, f, h, w = img.shape
    img3 = img.reshape(b, f, h * w)
    idx2 = index.reshape(b, h * w).astype(jnp.int32)
    return _face_pool(img3, idx2)


# speedup vs baseline: 4.8703x; 2.1371x over previous
import functools

import jax
import jax.numpy as jnp
from jax import lax
from jax.experimental import pallas as pl
from jax.experimental.pallas import tpu as pltpu
from jax.experimental.pallas import tpu_sc as plsc

B = 4
F = 192
HW = 224 * 224
S = 512
L = 16
NC, NS = 2, 16
NW = NC * NS
TPB = NW // B
FPT = F // TPB
K = 6
GRP = FPT // K
RH = 8
W = 224
CH = RH * W
NCH = 224 // RH
ACC = S * L + L


def _body(img_hbm, idx_hbm, out_hbm, addr_v,
          a0, a1, a2, a3, a4, a5,
          buf0, buf1, out_v, sem0, sem1):
    accs = (a0, a1, a2, a3, a4, a5)
    wid = lax.axis_index("s") * NC + lax.axis_index("c")
    b = wid // TPB
    f0 = (wid % TPB) * FPT
    lane = lax.broadcasted_iota(jnp.int32, (L,), 0)

    pltpu.sync_copy(idx_hbm.at[b], addr_v)

    def mk_addr(i, c):
        v = addr_v[pl.ds(i * L, L)]
        v = jnp.minimum(v, S)
        addr_v[pl.ds(i * L, L)] = jnp.where(
            v == 0, S * L + lane, (v - 1) * L + lane)
        return c

    lax.fori_loop(0, HW // L, mk_addr, 0)

    def src(g, c):
        return img_hbm.at[b, pl.ds(f0 + g * K, K), pl.ds(c * RH, RH), :]

    def process(cur, po, carry):
        def scat(i, cc):
            r = i // (W // L)
            w0 = (i % (W // L)) * L
            ad = addr_v[pl.ds(po + i * L, L)]
            gs = [plsc.load_gather(accs[k], [ad]) for k in range(K)]
            vs = [cur[k, r, pl.ds(w0, L)] for k in range(K)]
            ms = [jnp.maximum(g, v) for g, v in zip(gs, vs)]
            for k in range(K):
                plsc.store_scatter(accs[k], [ad], ms[k])
            return cc

        return lax.fori_loop(0, CH // L, scat, carry)

    perm_idx = tuple(jnp.bitwise_xor(lane, d) for d in (8, 4, 2, 1))
    lane_bit = tuple((lane & d) == 0 for d in (8, 4, 2, 1))

    pltpu.async_copy(src(0, 0), buf0, sem0)
    pltpu.async_copy(src(0, 1), buf1, sem1)

    def group_body(g, carry):
        def zero(i, cc):
            z = jnp.zeros((L,), jnp.float32)
            for acc in accs:
                acc[pl.ds(i * L, L)] = z
            return cc

        carry = lax.fori_loop(0, ACC // L, zero, carry)

        def pair(c2, cc):
            c = c2 * 2
            pltpu.make_async_copy(src(g, c), buf0, sem0).wait()
            cc = process(buf0, c * CH, cc)
            pltpu.async_copy(src(g, c + 2), buf0, sem0)
            pltpu.make_async_copy(src(g, c + 1), buf1, sem1).wait()
            cc = process(buf1, (c + 1) * CH, cc)
            pltpu.async_copy(src(g, c + 3), buf1, sem1)
            return cc

        carry = lax.fori_loop(0, NCH // 2 - 1, pair, carry)

        gn = jnp.minimum(g + 1, GRP - 1)
        pltpu.make_async_copy(src(g, NCH - 2), buf0, sem0).wait()
        carry = process(buf0, (NCH - 2) * CH, carry)
        pltpu.async_copy(src(gn, 0), buf0, sem0)
        pltpu.make_async_copy(src(g, NCH - 1), buf1, sem1).wait()
        carry = process(buf1, (NCH - 1) * CH, carry)
        pltpu.async_copy(src(gn, 1), buf1, sem1)

        for k in range(K):
            def red(j, cc, *, acc=accs[k]):
                rows = [acc[pl.ds(j * (L * L) + i * L, L)]
                        for i in range(L)]
                for lvl, d in enumerate((8, 4, 2, 1)):
                    half = len(rows) // 2
                    nxt = []
                    for i in range(half):
                        va, vb = rows[i], rows[i + half]
                        pa = va.at[perm_idx[lvl]].get(
                            mode="promise_in_bounds")
                        pb = vb.at[perm_idx[lvl]].get(
                            mode="promise_in_bounds")
                        nxt.append(jnp.where(lane_bit[lvl],
                                             jnp.maximum(va, pa),
                                             jnp.maximum(vb, pb)))
                    rows = nxt
                out_v[pl.ds(j * L, L)] = rows[0]
                return cc

            carry = lax.fori_loop(0, S // L, red, carry)
            pltpu.sync_copy(out_v, out_hbm.at[b, f0 + g * K + k])
        return carry

    lax.fori_loop(0, GRP, group_body, 0)
    pltpu.make_async_copy(src(0, 0), buf0, sem0).wait()
    pltpu.make_async_copy(src(0, 1), buf1, sem1).wait()


@jax.jit
def _face_pool(img4, idx2):
    mesh = plsc.VectorSubcoreMesh(core_axis_name="c", subcore_axis_name="s")
    return pl.kernel(
        _body,
        out_type=jax.ShapeDtypeStruct((B, F, S), jnp.float32),
        mesh=mesh,
        compiler_params=pltpu.CompilerParams(needs_layout_passes=False),
        scratch_types=[
            pltpu.VMEM((HW,), jnp.int32),
            pltpu.VMEM((ACC,), jnp.float32),
            pltpu.VMEM((ACC,), jnp.float32),
            pltpu.VMEM((ACC,), jnp.float32),
            pltpu.VMEM((ACC,), jnp.float32),
            pltpu.VMEM((ACC,), jnp.float32),
            pltpu.VMEM((ACC,), jnp.float32),
            pltpu.VMEM((K, RH, W), jnp.float32),
            pltpu.VMEM((K, RH, W), jnp.float32),
            pltpu.VMEM((S,), jnp.float32),
            pltpu.SemaphoreType.DMA,
            pltpu.SemaphoreType.DMA,
        ],
    )(img4, idx2)


def kernel(img, index, max_index):
    del max_index
    b, f, h, w = img.shape
    idx2 = index.reshape(b, h * w).astype(jnp.int32)
    return _face_pool(img, idx2)


# speedup vs baseline: 5.6421x; 1.1585x over previous
import functools

import jax
import jax.numpy as jnp
from jax import lax
from jax.experimental import pallas as pl
from jax.experimental.pallas import tpu as pltpu
from jax.experimental.pallas import tpu_sc as plsc

B = 4
F = 192
HW = 224 * 224
S = 512
L = 16
NC, NS = 2, 16
NW = NC * NS
TPB = NW // B
FPT = F // TPB
K = 6
GRP = FPT // K
RH = 8
W = 224
CH = RH * W
NCH = 224 // RH
ACC = S * L + L


def _body(img_hbm, idx_hbm, out_hbm, addr_v,
          a0, a1, a2, a3, a4, a5,
          buf0, buf1, out_v, sem0, sem1):
    accs = (a0, a1, a2, a3, a4, a5)
    wid = lax.axis_index("s") * NC + lax.axis_index("c")
    b = wid // TPB
    f0 = (wid % TPB) * FPT
    lane = lax.broadcasted_iota(jnp.int32, (L,), 0)

    pltpu.sync_copy(idx_hbm.at[b], addr_v)

    def mk_addr(i, c):
        v = addr_v[pl.ds(i * L, L)]
        v = jnp.minimum(v, S)
        addr_v[pl.ds(i * L, L)] = jnp.where(
            v == 0, S * L + lane, (v - 1) * L + lane)
        return c

    lax.fori_loop(0, HW // L, mk_addr, 0)

    def src(g, c):
        return img_hbm.at[b, pl.ds(f0 + g * K, K), pl.ds(c * RH, RH), :]

    def process(cur, po, carry):
        def ldstep(i):
            r = i // (W // L)
            w0 = (i % (W // L)) * L
            ad = addr_v[pl.ds(po + i * L, L)]
            vs = [cur[k, r, pl.ds(w0, L)] for k in range(K)]
            return ad, vs

        ad0, vs0 = ldstep(0)
        gs0 = [plsc.load_gather(accs[k], [ad0]) for k in range(K)]
        ms0 = tuple(jnp.maximum(g, v) for g, v in zip(gs0, vs0))

        def scat(i, st):
            cc, ad_p, ms_p = st
            ad, vs = ldstep(i)
            for k in range(K):
                plsc.store_scatter(accs[k], [ad_p], ms_p[k])
            gs = [plsc.load_gather(accs[k], [ad]) for k in range(K)]
            ms = tuple(jnp.maximum(g, v) for g, v in zip(gs, vs))
            return (cc, ad, ms)

        cc, ad_l, ms_l = lax.fori_loop(1, CH // L, scat, (carry, ad0, ms0))
        for k in range(K):
            plsc.store_scatter(accs[k], [ad_l], ms_l[k])
        return cc

    perm_idx = tuple(jnp.bitwise_xor(lane, d) for d in (8, 4, 2, 1))
    lane_bit = tuple((lane & d) == 0 for d in (8, 4, 2, 1))

    pltpu.async_copy(src(0, 0), buf0, sem0)
    pltpu.async_copy(src(0, 1), buf1, sem1)

    def group_body(g, carry):
        def zero(i, cc):
            z = jnp.zeros((L,), jnp.float32)
            for acc in accs:
                acc[pl.ds(i * L, L)] = z
            return cc

        carry = lax.fori_loop(0, ACC // L, zero, carry)

        def pair(c2, cc):
            c = c2 * 2
            pltpu.make_async_copy(src(g, c), buf0, sem0).wait()
            cc = process(buf0, c * CH, cc)
            pltpu.async_copy(src(g, c + 2), buf0, sem0)
            pltpu.make_async_copy(src(g, c + 1), buf1, sem1).wait()
            cc = process(buf1, (c + 1) * CH, cc)
            pltpu.async_copy(src(g, c + 3), buf1, sem1)
            return cc

        carry = lax.fori_loop(0, NCH // 2 - 1, pair, carry)

        gn = jnp.minimum(g + 1, GRP - 1)
        pltpu.make_async_copy(src(g, NCH - 2), buf0, sem0).wait()
        carry = process(buf0, (NCH - 2) * CH, carry)
        pltpu.async_copy(src(gn, 0), buf0, sem0)
        pltpu.make_async_copy(src(g, NCH - 1), buf1, sem1).wait()
        carry = process(buf1, (NCH - 1) * CH, carry)
        pltpu.async_copy(src(gn, 1), buf1, sem1)

        for k in range(K):
            def red(j, cc, *, acc=accs[k]):
                rows = [acc[pl.ds(j * (L * L) + i * L, L)]
                        for i in range(L)]
                for lvl, d in enumerate((8, 4, 2, 1)):
                    half = len(rows) // 2
                    nxt = []
                    for i in range(half):
                        va, vb = rows[i], rows[i + half]
                        pa = va.at[perm_idx[lvl]].get(
                            mode="promise_in_bounds")
                        pb = vb.at[perm_idx[lvl]].get(
                            mode="promise_in_bounds")
                        nxt.append(jnp.where(lane_bit[lvl],
                                             jnp.maximum(va, pa),
                                             jnp.maximum(vb, pb)))
                    rows = nxt
                out_v[pl.ds(j * L, L)] = rows[0]
                return cc

            carry = lax.fori_loop(0, S // L, red, carry)
            pltpu.sync_copy(out_v, out_hbm.at[b, f0 + g * K + k])
        return carry

    lax.fori_loop(0, GRP, group_body, 0)
    pltpu.make_async_copy(src(0, 0), buf0, sem0).wait()
    pltpu.make_async_copy(src(0, 1), buf1, sem1).wait()


@jax.jit
def _face_pool(img4, idx2):
    mesh = plsc.VectorSubcoreMesh(core_axis_name="c", subcore_axis_name="s")
    return pl.kernel(
        _body,
        out_type=jax.ShapeDtypeStruct((B, F, S), jnp.float32),
        mesh=mesh,
        compiler_params=pltpu.CompilerParams(needs_layout_passes=False),
        scratch_types=[
            pltpu.VMEM((HW,), jnp.int32),
            pltpu.VMEM((ACC,), jnp.float32),
            pltpu.VMEM((ACC,), jnp.float32),
            pltpu.VMEM((ACC,), jnp.float32),
            pltpu.VMEM((ACC,), jnp.float32),
            pltpu.VMEM((ACC,), jnp.float32),
            pltpu.VMEM((ACC,), jnp.float32),
            pltpu.VMEM((K, RH, W), jnp.float32),
            pltpu.VMEM((K, RH, W), jnp.float32),
            pltpu.VMEM((S,), jnp.float32),
            pltpu.SemaphoreType.DMA,
            pltpu.SemaphoreType.DMA,
        ],
    )(img4, idx2)


def kernel(img, index, max_index):
    del max_index
    b, f, h, w = img.shape
    idx2 = index.reshape(b, h * w).astype(jnp.int32)
    return _face_pool(img, idx2)


# speedup vs baseline: 5.7806x; 1.0245x over previous
import functools

import jax
import jax.numpy as jnp
from jax import lax
from jax.experimental import pallas as pl
from jax.experimental.pallas import tpu as pltpu
from jax.experimental.pallas import tpu_sc as plsc

B = 4
F = 192
HW = 224 * 224
S = 512
L = 16
NC, NS = 2, 16
NW = NC * NS
TPB = NW // B
FPT = F // TPB
K = 6
GRP = FPT // K
RH = 8
W = 224
CH = RH * W
NCH = 224 // RH
ACC = S * L + L


def _body(img_hbm, idx_hbm, out_hbm, addr_v,
          a0, a1, a2, a3, a4, a5,
          buf0, buf1, out_v, sem0, sem1):
    accs = (a0, a1, a2, a3, a4, a5)
    wid = lax.axis_index("s") * NC + lax.axis_index("c")
    b = wid // TPB
    f0 = (wid % TPB) * FPT
    lane = lax.broadcasted_iota(jnp.int32, (L,), 0)

    pltpu.sync_copy(idx_hbm.at[b], addr_v)

    def mk_addr(i, c):
        v = addr_v[pl.ds(i * L, L)]
        v = jnp.minimum(v, S)
        addr_v[pl.ds(i * L, L)] = jnp.where(
            v == 0, S * L + lane, (v - 1) * L + lane)
        return c

    lax.fori_loop(0, HW // L, mk_addr, 0)

    def src(g, c):
        return img_hbm.at[b, pl.ds(f0 + g * K, K), pl.ds(c * RH, RH), :]

    def process(cur, ro, carry):
        def ldstep(i):
            r = i // (W // L)
            w0 = (i % (W // L)) * L
            ad = addr_v[pl.ds(ro * W + i * L, L)]
            vs = [cur[k, r, pl.ds(w0, L)] for k in range(K)]
            return ad, vs

        ad0, vs0 = ldstep(0)
        gs0 = [plsc.load_gather(accs[k], [ad0]) for k in range(K)]
        ms0 = tuple(jnp.maximum(g, v) for g, v in zip(gs0, vs0))

        def scat(i, st):
            cc, ad_p, ms_p = st
            ad, vs = ldstep(i)
            for k in range(K):
                plsc.store_scatter(accs[k], [ad_p], ms_p[k])
            gs = [plsc.load_gather(accs[k], [ad]) for k in range(K)]
            ms = tuple(jnp.maximum(g, v) for g, v in zip(gs, vs))
            return (cc, ad, ms)

        cc, ad_l, ms_l = lax.fori_loop(1, CH // L, scat, (carry, ad0, ms0))
        for k in range(K):
            plsc.store_scatter(accs[k], [ad_l], ms_l[k])
        return cc

    perm_idx = tuple(jnp.bitwise_xor(lane, d) for d in (8, 4, 2, 1))
    lane_bit = tuple((lane & d) == 0 for d in (8, 4, 2, 1))

    pltpu.async_copy(src(0, 0), buf0, sem0)
    pltpu.async_copy(src(0, 1), buf1, sem1)

    def zero(i, cc):
        z = jnp.zeros((L,), jnp.float32)
        for acc in accs:
            acc[pl.ds(i * L, L)] = z
        return cc

    lax.fori_loop(0, ACC // L, zero, 0)

    def group_body(g, carry):
        def pair(c2, cc):
            c = c2 * 2
            pltpu.make_async_copy(src(g, c), buf0, sem0).wait()
            cc = process(buf0, c * RH, cc)
            pltpu.async_copy(src(g, c + 2), buf0, sem0)
            pltpu.make_async_copy(src(g, c + 1), buf1, sem1).wait()
            cc = process(buf1, (c + 1) * RH, cc)
            pltpu.async_copy(src(g, c + 3), buf1, sem1)
            return cc

        carry = lax.fori_loop(0, NCH // 2 - 1, pair, carry)

        gn = jnp.minimum(g + 1, GRP - 1)
        pltpu.make_async_copy(src(g, NCH - 2), buf0, sem0).wait()
        carry = process(buf0, (NCH - 2) * RH, carry)
        pltpu.async_copy(src(gn, 0), buf0, sem0)
        pltpu.make_async_copy(src(g, NCH - 1), buf1, sem1).wait()
        carry = process(buf1, (NCH - 1) * RH, carry)
        pltpu.async_copy(src(gn, 1), buf1, sem1)

        for k in range(K):
            def red(j, cc, *, acc=accs[k]):
                rows = [acc[pl.ds(j * (L * L) + i * L, L)]
                        for i in range(L)]
                z = jnp.zeros((L,), jnp.float32)
                for i in range(L):
                    acc[pl.ds(j * (L * L) + i * L, L)] = z
                for lvl, d in enumerate((8, 4, 2, 1)):
                    half = len(rows) // 2
                    nxt = []
                    for i in range(half):
                        va, vb = rows[i], rows[i + half]
                        pa = va.at[perm_idx[lvl]].get(
                            mode="promise_in_bounds")
                        pb = vb.at[perm_idx[lvl]].get(
                            mode="promise_in_bounds")
                        nxt.append(jnp.where(lane_bit[lvl],
                                             jnp.maximum(va, pa),
                                             jnp.maximum(vb, pb)))
                    rows = nxt
                out_v[pl.ds(j * L, L)] = rows[0]
                return cc

            carry = lax.fori_loop(0, S // L, red, carry)
            pltpu.sync_copy(out_v, out_hbm.at[b, f0 + g * K + k])
        return carry

    lax.fori_loop(0, GRP, group_body, 0)
    pltpu.make_async_copy(src(0, 0), buf0, sem0).wait()
    pltpu.make_async_copy(src(0, 1), buf1, sem1).wait()


@jax.jit
def _face_pool(img4, idx2):
    mesh = plsc.VectorSubcoreMesh(core_axis_name="c", subcore_axis_name="s")
    return pl.kernel(
        _body,
        out_type=jax.ShapeDtypeStruct((B, F, S), jnp.float32),
        mesh=mesh,
        compiler_params=pltpu.CompilerParams(needs_layout_passes=False),
        scratch_types=[
            pltpu.VMEM((HW,), jnp.int32),
            pltpu.VMEM((ACC,), jnp.float32),
            pltpu.VMEM((ACC,), jnp.float32),
            pltpu.VMEM((ACC,), jnp.float32),
            pltpu.VMEM((ACC,), jnp.float32),
            pltpu.VMEM((ACC,), jnp.float32),
            pltpu.VMEM((ACC,), jnp.float32),
            pltpu.VMEM((K, RH, W), jnp.float32),
            pltpu.VMEM((K, RH, W), jnp.float32),
            pltpu.VMEM((S,), jnp.float32),
            pltpu.SemaphoreType.DMA,
            pltpu.SemaphoreType.DMA,
        ],
    )(img4, idx2)


def kernel(img, index, max_index):
    del max_index
    b, f, h, w = img.shape
    idx2 = index.reshape(b, h * w).astype(jnp.int32)
    return _face_pool(img, idx2)
